# Initial kernel scaffold; baseline (speedup 1.0000x reference)
import jax
import jax.numpy as jnp
from jax.experimental import pallas as pl


def kernel(x, edge_index, edge_values, positions, W_qkv, b_qkv, W_proj, b_proj, W_gate, b_gate):
    raise NotImplementedError("write your pallas kernel here")


# speedup vs baseline: 1.3449x; 1.3449x over previous
import functools

import jax
import jax.numpy as jnp
from jax import lax
from jax.experimental import pallas as pl
from jax.experimental.pallas import tpu as pltpu

_DIM = 256
_BLOCK = 50
_HEADS = 8
_HD = _DIM // _HEADS
_SCALE = _HD ** -0.5
_QP = 56
_KP = 64
_G = 4
_NEG = -1e30


def _attn_body(x_ref, m_ref, f0_ref, f1_ref, f2_ref, f3_ref,
               wqkv_ref, bqkv_ref, wproj_ref, bproj_ref, wg_ref, bg_ref,
               out_ref, kv_s, attn_s):
    xb = x_ref[0]
    mean = jnp.sum(xb, axis=1) * (1.0 / _BLOCK)
    zrows = jnp.zeros((_KP - _BLOCK - 1, _DIM), dtype=jnp.float32)
    for g in range(_G):
        kv_s[pl.ds(g * _KP, _BLOCK), :] = xb[g]
        kv_s[pl.ds(g * _KP + _BLOCK, 1), :] = mean[g:g + 1]
        kv_s[pl.ds(g * _KP + _BLOCK + 1, _KP - _BLOCK - 1), :] = zrows
    qkv = jnp.dot(kv_s[...], wqkv_ref[...],
                  preferred_element_type=jnp.float32) + bqkv_ref[...]
    kidx = lax.broadcasted_iota(jnp.int32, (_QP, _KP), 1)
    qidx = lax.broadcasted_iota(jnp.int32, (_QP, _KP), 0)
    dc = (kidx == _BLOCK) | (kidx == qidx)
    for g in range(_G):
        m_g = m_ref[g]
        cond = (kidx < _BLOCK + 1) & (dc | (m_g != 0.0))
        bias = jnp.where(dc, 1.0, f3_ref[g])
        f0m = jnp.where(dc, 0.0, f0_ref[g])
        f1m = jnp.where(dc, 0.0, f1_ref[g])
        f2m = jnp.where(dc, 0.0, f2_ref[g])
        q_g = qkv[g * _KP:g * _KP + _QP, 0:_DIM]
        k_g = qkv[g * _KP:(g + 1) * _KP, _DIM:2 * _DIM]
        v_g = qkv[g * _KP:(g + 1) * _KP, 2 * _DIM:3 * _DIM]
        outs = []
        for h in range(_HEADS):
            qh = q_g[:, h * _HD:(h + 1) * _HD]
            kh = k_g[:, h * _HD:(h + 1) * _HD]
            vh = v_g[:, h * _HD:(h + 1) * _HD]
            s = lax.dot_general(qh, kh, (((1,), (1,)), ((), ())),
                                preferred_element_type=jnp.float32)
            s = s * _SCALE + bias
            s = jnp.where(cond, s, _NEG)
            mrow = jnp.max(s, axis=1, keepdims=True)
            e = jnp.exp(s - mrow)
            probs = e / jnp.sum(e, axis=1, keepdims=True)
            lew = (f0m * wg_ref[0, h] + f1m * wg_ref[1, h]
                   + f2m * wg_ref[2, h] + bias * wg_ref[3, h] + bg_ref[0, h])
            comb = probs + jnp.where(cond, lew, 0.0)
            outs.append(lax.dot_general(comb, vh, (((1,), (0,)), ((), ())),
                                        preferred_element_type=jnp.float32))
        attn_s[pl.ds(g * _QP, _QP), :] = jnp.concatenate(outs, axis=1)
    proj = jnp.dot(attn_s[...], wproj_ref[...],
                   preferred_element_type=jnp.float32) + bproj_ref[...]
    for g in range(_G):
        out_ref[0, g] = proj[g * _QP:g * _QP + _BLOCK, :]


def _block_attention(x4, m, f0, f1, f2, f3, W_qkv, b_qkv, W_proj, b_proj,
                     W_gate, b_gate):
    Bb, nb, _, _ = x4.shape
    grid = (Bb, nb // _G)
    bspec_x = pl.BlockSpec((1, _G, _BLOCK, _DIM), lambda b, g: (b, g, 0, 0))
    bspec_m = pl.BlockSpec((_G, _QP, _KP), lambda b, g: (g, 0, 0))
    bspec_full = lambda shape: pl.BlockSpec(shape, lambda b, g: (0,) * len(shape))
    bspec_smem = pl.BlockSpec(memory_space=pltpu.SMEM)
    return pl.pallas_call(
        _attn_body,
        grid=grid,
        in_specs=[bspec_x, bspec_m, bspec_m, bspec_m, bspec_m, bspec_m,
                  bspec_full((_DIM, 3 * _DIM)), bspec_full((1, 3 * _DIM)),
                  bspec_full((_DIM, _DIM)), bspec_full((1, _DIM)),
                  bspec_smem, bspec_smem],
        out_specs=pl.BlockSpec((1, _G, _BLOCK, _DIM), lambda b, g: (b, g, 0, 0)),
        out_shape=jax.ShapeDtypeStruct((Bb, nb, _BLOCK, _DIM), jnp.float32),
        scratch_shapes=[pltpu.VMEM((_G * _KP, _DIM), jnp.float32),
                        pltpu.VMEM((_G * _QP, _DIM), jnp.float32)],
    )(x4, m, f0, f1, f2, f3, W_qkv, b_qkv, W_proj, b_proj, W_gate, b_gate)


def _build_edge_buffers(edge_index, edge_values, positions, nb):
    rows = edge_index[0]
    cols = edge_index[1]
    br = rows // _BLOCK
    bc = cols // _BLOCK
    bl = jnp.where(br == bc, br, nb)
    rl = rows % _BLOCK
    cl = cols % _BLOCK
    dx = positions[cols] - positions[rows]
    zero = jnp.zeros((nb, _QP, _KP), jnp.float32)
    m = zero.at[bl, rl, cl].add(1.0, mode='drop')
    f0 = zero.at[bl, rl, cl].add(dx[:, 0], mode='drop')
    f1 = zero.at[bl, rl, cl].add(dx[:, 1], mode='drop')
    f2 = zero.at[bl, rl, cl].add(dx[:, 2], mode='drop')
    f3 = zero.at[bl, rl, cl].add(edge_values, mode='drop')
    return m, f0, f1, f2, f3


def kernel(x, edge_index, edge_values, positions, W_qkv, b_qkv, W_proj,
           b_proj, W_gate, b_gate):
    Bb, N, C = x.shape
    nb = N // _BLOCK
    x4 = x.reshape(Bb, nb, _BLOCK, C)
    m, f0, f1, f2, f3 = _build_edge_buffers(edge_index, edge_values,
                                            positions, nb)
    out = _block_attention(x4, m, f0, f1, f2, f3,
                           W_qkv, b_qkv.reshape(1, -1),
                           W_proj, b_proj.reshape(1, -1),
                           W_gate, b_gate.reshape(1, -1))
    return out.reshape(Bb, N, C)
